# baseline (speedup 1.0000x reference)
import jax
import jax.numpy as jnp
from jax.experimental import pallas as pl
from jax.experimental.pallas import tpu as pltpu

D = 512
NN = 4096
BLK = 1024
NBLK = NN // BLK


def _body(cell_ref, idx_ref, ns_ref, cs_ref, Wm_ref, bm_ref,
          Wl_hbm, bl_ref, Wu_hbm, bu_ref, W1_hbm, b1_ref, W2_hbm, b2_ref,
          Wg_ref, bg_ref, out_ref, acc_ref, cnt_ref,
          Wl_v, Wu_v, W1_v, W2_v, sl, su, s1, s2):
    i = pl.program_id(0)

    @pl.when(i == 0)
    def _init():
        acc_ref[...] = jnp.zeros_like(acc_ref)
        cnt_ref[0] = 0.0
        cnt_ref[1] = 0.0
        cnt_ref[2] = 0.0
        pltpu.make_async_copy(Wl_hbm, Wl_v, sl).start()
        pltpu.make_async_copy(Wu_hbm, Wu_v, su).start()
        pltpu.make_async_copy(W1_hbm, W1_v, s1).start()
        pltpu.make_async_copy(W2_hbm, W2_v, s2).start()

    cell = cell_ref[0]
    cx = (cell // 729).astype(jnp.float32)
    cy = ((cell // 27) % 27).astype(jnp.float32)
    cz = (cell % 27).astype(jnp.float32)

    idx = idx_ref[0].astype(jnp.float32)
    nx = jnp.floor(idx * (1.0 / 729.0))
    r = idx - 729.0 * nx
    ny = jnp.floor(r * (1.0 / 27.0))
    nz = r - 27.0 * ny
    d2 = (nx - cx) ** 2 + (ny - cy) ** 2 + (nz - cz) ** 2
    local_m = jnp.where(d2 <= 3.24, 1.0, 0.0)
    dist_m = jnp.where(d2 > 36.0, 1.0, 0.0)
    func_m = 1.0 - local_m - dist_m

    cnt_ref[0] += jnp.sum(local_m)
    cnt_ref[1] += jnp.sum(func_m)
    cnt_ref[2] += jnp.sum(dist_m)

    ns = ns_ref[...]
    nsb = ns.astype(jnp.bfloat16)

    row = jax.lax.broadcasted_iota(jnp.int32, (8, BLK), 0)
    M = jnp.where(row == 0, local_m, jnp.where(row == 1, func_m,
        jnp.where(row == 2, dist_m, 0.0)))
    Mt = jnp.transpose(M, (1, 0))
    lm_col = Mt[:, 0:1]
    fm_col = Mt[:, 1:2]
    dm_col = Mt[:, 2:3]

    acc_ref[0:1, :] += jnp.sum(ns * lm_col, axis=0, keepdims=True)
    acc_ref[1:2, :] += jnp.sum(ns * dm_col, axis=0, keepdims=True)
    acc_ref[2:3, :] += jnp.sum(ns, axis=0, keepdims=True)

    t = jnp.tanh(jax.lax.dot_general(
        nsb, Wm_ref[...].astype(jnp.bfloat16), (((1,), (0,)), ((), ())),
        preferred_element_type=jnp.float32) + bm_ref[...])
    acc_ref[8:9, :] += jnp.sum(t * fm_col, axis=0, keepdims=True)

    @pl.when(i == NBLK - 1)
    def _final():
        pltpu.make_async_copy(Wl_hbm, Wl_v, sl).wait()
        pltpu.make_async_copy(Wu_hbm, Wu_v, su).wait()
        pltpu.make_async_copy(W1_hbm, W1_v, s1).wait()
        pltpu.make_async_copy(W2_hbm, W2_v, s2).wait()
        lc = jnp.maximum(cnt_ref[0], 1.0)
        fc = jnp.maximum(cnt_ref[1], 1.0)
        dc = jnp.maximum(cnt_ref[2], 1.0)
        local_agg = acc_ref[0:1, :] / lc
        dist_agg = acc_ref[1:2, :] / dc
        all_agg = acc_ref[2:3, :] * (1.0 / NN)
        func_agg = acc_ref[8:9, :] / fc

        cs = cs_ref[...]

        def mm(a, w):
            return jax.lax.dot_general(a, w, (((1,), (0,)), ((), ())),
                                       preferred_element_type=jnp.float32)

        xl = jnp.concatenate([cs, local_agg], axis=1)
        local_out = jnp.tanh(mm(xl, Wl_v[...]) + bl_ref[...])

        xf = jnp.concatenate([cs, func_agg], axis=1)
        func_out = jnp.tanh(mm(xf, Wu_v[...]) + bu_ref[...])

        z = cs
        for _ in range(3):
            h = jnp.tanh(mm(jnp.concatenate([z, dist_agg], axis=1), W1_v[...])
                         + b1_ref[...])
            z = z + 0.3 * (mm(h, W2_v[...]) + b2_ref[...])

        logits = mm(jnp.concatenate([cs, all_agg], axis=1), Wg_ref[...]) + bg_ref[...]
        m = jnp.max(logits, axis=1, keepdims=True)
        e = jnp.exp(logits - m)
        g = e / jnp.sum(e, axis=1, keepdims=True)

        out_ref[...] = (g[:, 0:1] * local_out + g[:, 1:2] * func_out
                        + g[:, 2:3] * z)


def kernel(current_state, neighbor_states, cell_idx, neighbor_indices,
           W_local, b_local, W_msg, b_msg, W_upd, b_upd,
           W_cnf1, b_cnf1, W_cnf2, b_cnf2, W_gate, b_gate):
    cell = jnp.reshape(jnp.asarray(cell_idx, dtype=jnp.int32), (1,))
    idx3 = jnp.reshape(neighbor_indices.astype(jnp.int32), (NBLK, 1, BLK))
    cs = jnp.reshape(current_state, (1, D))

    full = lambda shape: pl.BlockSpec(shape, lambda i: (0,) * len(shape))
    out = pl.pallas_call(
        _body,
        grid=(NBLK,),
        in_specs=[
            pl.BlockSpec(memory_space=pltpu.SMEM),
            pl.BlockSpec((1, 1, BLK), lambda i: (i, 0, 0)),
            pl.BlockSpec((BLK, D), lambda i: (i, 0)),
            full((1, D)),
            full((D, D)),
            full((1, D)),
            pl.BlockSpec(memory_space=pl.ANY),
            full((1, D)),
            pl.BlockSpec(memory_space=pl.ANY),
            full((1, D)),
            pl.BlockSpec(memory_space=pl.ANY),
            full((1, 2 * D)),
            pl.BlockSpec(memory_space=pl.ANY),
            full((1, D)),
            full((2 * D, 3)),
            full((1, 3)),
        ],
        out_specs=pl.BlockSpec((1, D), lambda i: (0, 0)),
        out_shape=jax.ShapeDtypeStruct((1, D), jnp.float32),
        scratch_shapes=[
            pltpu.VMEM((9, D), jnp.float32),
            pltpu.SMEM((4,), jnp.float32),
            pltpu.VMEM((2 * D, D), jnp.float32),
            pltpu.VMEM((2 * D, D), jnp.float32),
            pltpu.VMEM((2 * D, 2 * D), jnp.float32),
            pltpu.VMEM((2 * D, D), jnp.float32),
            pltpu.SemaphoreType.DMA,
            pltpu.SemaphoreType.DMA,
            pltpu.SemaphoreType.DMA,
            pltpu.SemaphoreType.DMA,
        ],
    )(cell, idx3, neighbor_states, cs, W_msg, jnp.reshape(b_msg, (1, D)),
      W_local, jnp.reshape(b_local, (1, D)), W_upd, jnp.reshape(b_upd, (1, D)),
      W_cnf1, jnp.reshape(b_cnf1, (1, 2 * D)), W_cnf2, jnp.reshape(b_cnf2, (1, D)),
      W_gate, jnp.reshape(b_gate, (1, 3)))
    return jnp.reshape(out, (D,))


# speedup vs baseline: 1.0755x; 1.0755x over previous
import jax
import jax.numpy as jnp
from jax.experimental import pallas as pl
from jax.experimental.pallas import tpu as pltpu

D = 512
NN = 4096
BLK = 1024
NBLK = NN // BLK


def _body(cell_ref, idx_ref, ns_ref, cs_ref, Wm_ref, bm_ref,
          Wl_hbm, bl_ref, Wu_hbm, bu_ref, W1_hbm, b1_ref, W2_hbm, b2_ref,
          Wg_ref, bg_ref, out_ref, acc_ref, cnt_ref,
          Wl_v, Wu_v, W1_v, W2_v, sl, su, s1, s2):
    i = pl.program_id(0)

    @pl.when(i == 0)
    def _init():
        acc_ref[...] = jnp.zeros_like(acc_ref)
        cnt_ref[0] = 0.0
        cnt_ref[1] = 0.0
        cnt_ref[2] = 0.0

    @pl.when(i == 1)
    def _start_lu():
        pltpu.make_async_copy(Wl_hbm, Wl_v, sl).start()
        pltpu.make_async_copy(Wu_hbm, Wu_v, su).start()

    @pl.when(i == 2)
    def _start_12():
        pltpu.make_async_copy(W1_hbm, W1_v, s1).start()
        pltpu.make_async_copy(W2_hbm, W2_v, s2).start()

    cell = cell_ref[0]
    cx = (cell // 729).astype(jnp.float32)
    cy = ((cell // 27) % 27).astype(jnp.float32)
    cz = (cell % 27).astype(jnp.float32)

    idx = idx_ref[0].astype(jnp.float32)
    nx = jnp.floor(idx * (1.0 / 729.0))
    r = idx - 729.0 * nx
    ny = jnp.floor(r * (1.0 / 27.0))
    nz = r - 27.0 * ny
    d2 = (nx - cx) ** 2 + (ny - cy) ** 2 + (nz - cz) ** 2
    local_m = jnp.where(d2 <= 3.24, 1.0, 0.0)
    dist_m = jnp.where(d2 > 36.0, 1.0, 0.0)
    func_m = 1.0 - local_m - dist_m

    cnt_ref[0] += jnp.sum(local_m)
    cnt_ref[1] += jnp.sum(func_m)
    cnt_ref[2] += jnp.sum(dist_m)

    ns = ns_ref[...]
    nsb = ns.astype(jnp.bfloat16)

    row = jax.lax.broadcasted_iota(jnp.int32, (8, BLK), 0)
    M = jnp.where(row == 0, local_m, jnp.where(row == 1, func_m,
        jnp.where(row == 2, dist_m, 0.0)))
    Mt = jnp.transpose(M, (1, 0))
    lm_col = Mt[:, 0:1]
    fm_col = Mt[:, 1:2]
    dm_col = Mt[:, 2:3]

    acc_ref[0:1, :] += jnp.sum(ns * lm_col, axis=0, keepdims=True)
    acc_ref[1:2, :] += jnp.sum(ns * dm_col, axis=0, keepdims=True)
    acc_ref[2:3, :] += jnp.sum(ns, axis=0, keepdims=True)

    t = jnp.tanh(jax.lax.dot_general(
        nsb, Wm_ref[...].astype(jnp.bfloat16), (((1,), (0,)), ((), ())),
        preferred_element_type=jnp.float32) + bm_ref[...])
    acc_ref[8:9, :] += jnp.sum(t * fm_col, axis=0, keepdims=True)

    @pl.when(i == NBLK - 1)
    def _final():
        pltpu.make_async_copy(Wl_hbm, Wl_v, sl).wait()
        pltpu.make_async_copy(Wu_hbm, Wu_v, su).wait()
        pltpu.make_async_copy(W1_hbm, W1_v, s1).wait()
        pltpu.make_async_copy(W2_hbm, W2_v, s2).wait()
        lc = jnp.maximum(cnt_ref[0], 1.0)
        fc = jnp.maximum(cnt_ref[1], 1.0)
        dc = jnp.maximum(cnt_ref[2], 1.0)
        local_agg = acc_ref[0:1, :] / lc
        dist_agg = acc_ref[1:2, :] / dc
        all_agg = acc_ref[2:3, :] * (1.0 / NN)
        func_agg = acc_ref[8:9, :] / fc

        cs = cs_ref[...]

        def mm(a, w):
            return jax.lax.dot_general(a, w, (((1,), (0,)), ((), ())),
                                       preferred_element_type=jnp.float32)

        xl = jnp.concatenate([cs, local_agg], axis=1)
        local_out = jnp.tanh(mm(xl, Wl_v[...]) + bl_ref[...])

        xf = jnp.concatenate([cs, func_agg], axis=1)
        func_out = jnp.tanh(mm(xf, Wu_v[...]) + bu_ref[...])

        z = cs
        for _ in range(3):
            h = jnp.tanh(mm(jnp.concatenate([z, dist_agg], axis=1), W1_v[...])
                         + b1_ref[...])
            z = z + 0.3 * (mm(h, W2_v[...]) + b2_ref[...])

        logits = mm(jnp.concatenate([cs, all_agg], axis=1), Wg_ref[...]) + bg_ref[...]
        m = jnp.max(logits, axis=1, keepdims=True)
        e = jnp.exp(logits - m)
        g = e / jnp.sum(e, axis=1, keepdims=True)

        out_ref[...] = (g[:, 0:1] * local_out + g[:, 1:2] * func_out
                        + g[:, 2:3] * z)


def kernel(current_state, neighbor_states, cell_idx, neighbor_indices,
           W_local, b_local, W_msg, b_msg, W_upd, b_upd,
           W_cnf1, b_cnf1, W_cnf2, b_cnf2, W_gate, b_gate):
    cell = jnp.reshape(jnp.asarray(cell_idx, dtype=jnp.int32), (1,))
    idx3 = jnp.reshape(neighbor_indices.astype(jnp.int32), (NBLK, 1, BLK))
    cs = jnp.reshape(current_state, (1, D))

    full = lambda shape: pl.BlockSpec(shape, lambda i: (0,) * len(shape))
    out = pl.pallas_call(
        _body,
        grid=(NBLK,),
        in_specs=[
            pl.BlockSpec(memory_space=pltpu.SMEM),
            pl.BlockSpec((1, 1, BLK), lambda i: (i, 0, 0)),
            pl.BlockSpec((BLK, D), lambda i: (i, 0)),
            full((1, D)),
            full((D, D)),
            full((1, D)),
            pl.BlockSpec(memory_space=pl.ANY),
            full((1, D)),
            pl.BlockSpec(memory_space=pl.ANY),
            full((1, D)),
            pl.BlockSpec(memory_space=pl.ANY),
            full((1, 2 * D)),
            pl.BlockSpec(memory_space=pl.ANY),
            full((1, D)),
            full((2 * D, 3)),
            full((1, 3)),
        ],
        out_specs=pl.BlockSpec((1, D), lambda i: (0, 0)),
        out_shape=jax.ShapeDtypeStruct((1, D), jnp.float32),
        scratch_shapes=[
            pltpu.VMEM((9, D), jnp.float32),
            pltpu.SMEM((4,), jnp.float32),
            pltpu.VMEM((2 * D, D), jnp.float32),
            pltpu.VMEM((2 * D, D), jnp.float32),
            pltpu.VMEM((2 * D, 2 * D), jnp.float32),
            pltpu.VMEM((2 * D, D), jnp.float32),
            pltpu.SemaphoreType.DMA,
            pltpu.SemaphoreType.DMA,
            pltpu.SemaphoreType.DMA,
            pltpu.SemaphoreType.DMA,
        ],
    )(cell, idx3, neighbor_states, cs, W_msg, jnp.reshape(b_msg, (1, D)),
      W_local, jnp.reshape(b_local, (1, D)), W_upd, jnp.reshape(b_upd, (1, D)),
      W_cnf1, jnp.reshape(b_cnf1, (1, 2 * D)), W_cnf2, jnp.reshape(b_cnf2, (1, D)),
      W_gate, jnp.reshape(b_gate, (1, 3)))
    return jnp.reshape(out, (D,))
